# baseline (speedup 1.0000x reference)
import functools

import jax
import jax.numpy as jnp
from jax import lax
from jax.experimental import pallas as pl
from jax.experimental.pallas import tpu as pltpu
from jax.experimental.pallas import tpu_sc as plsc

B = 16384
D = 32
M = 28
C_NUM = 64
TAU = 10.0

NUM_CORES = 2
NUM_SUBCORES = 16
NW = NUM_CORES * NUM_SUBCORES
BPW = B // NW

ROWS = 2048
NB = B // ROWS


def _sc_gather(ut_tab, it_tab, iu, ii):
    mesh = plsc.VectorSubcoreMesh(core_axis_name="c", subcore_axis_name="s")

    @functools.partial(
        pl.kernel,
        mesh=mesh,
        out_type=(
            jax.ShapeDtypeStruct((D, B), jnp.float32),
            jax.ShapeDtypeStruct((D, B), jnp.float32),
        ),
        scratch_types=[
            pltpu.VMEM((BPW,), jnp.int32),
            pltpu.VMEM((BPW,), jnp.int32),
            pltpu.VMEM((D, BPW), jnp.float32),
            pltpu.VMEM((D, BPW), jnp.float32),
            pltpu.SemaphoreType.DMA,
        ],
        compiler_params=pltpu.CompilerParams(use_tc_tiling_on_sc=False),
    )
    def k(ut_hbm, it_hbm, iu_hbm, ii_hbm, ut_out, it_out,
          iu_v, ii_v, du_v, di_v, sem):
        wid = lax.axis_index("s") * NUM_CORES + lax.axis_index("c")
        base = wid * BPW
        pltpu.sync_copy(iu_hbm.at[pl.ds(base, BPW)], iu_v)
        pltpu.sync_copy(ii_hbm.at[pl.ds(base, BPW)], ii_v)
        copies = []
        for d in range(D):
            copies.append(pltpu.async_copy(ut_hbm.at[d].at[iu_v], du_v.at[d], sem))
            copies.append(pltpu.async_copy(it_hbm.at[d].at[ii_v], di_v.at[d], sem))
        for c in copies:
            c.wait()
        pltpu.sync_copy(du_v, ut_out.at[:, pl.ds(base, BPW)])
        pltpu.sync_copy(di_v, it_out.at[:, pl.ds(base, BPW)])

    return k(ut_tab, it_tab, iu, ii)


def _tc_body(ut_ref, it_ref, gt_ref, ct_ref, w1t_ref, w2t_ref, out_ref):
    hi = lax.Precision.HIGHEST
    ut = ut_ref[...]
    it = it_ref[...]
    gt = gt_ref[...]
    ct = ct_ref[...]
    a = lax.dot_general(w1t_ref[...], ct, (((0,), (0,)), ((), ())),
                        precision=hi, preferred_element_type=jnp.float32)
    cw2t = lax.dot_general(w2t_ref[...], ct, (((1,), (0,)), ((), ())),
                           precision=hi, preferred_element_type=jnp.float32)
    st = lax.dot_general(a, ut, (((0,), (0,)), ((), ())),
                         precision=hi, preferred_element_type=jnp.float32)
    tt = lax.dot_general(cw2t, it, (((0,), (0,)), ((), ())),
                         precision=hi, preferred_element_type=jnp.float32)
    l = (st + gt) / TAU
    m1 = jnp.max(l, axis=0, keepdims=True)
    iota = lax.broadcasted_iota(jnp.int32, l.shape, 0)
    idx1 = jnp.min(jnp.where(l == m1, iota, C_NUM), axis=0, keepdims=True)
    oh1 = iota == idx1
    l2 = jnp.where(oh1, -jnp.inf, l)
    m2 = jnp.max(l2, axis=0, keepdims=True)
    idx2 = jnp.min(jnp.where(l2 == m2, iota, C_NUM), axis=0, keepdims=True)
    oh2 = iota == idx2
    z = jnp.sum(jnp.exp(l - m1), axis=0, keepdims=True)
    t1 = jnp.sum(jnp.where(oh1, tt, 0.0), axis=0, keepdims=True)
    t2 = jnp.sum(jnp.where(oh2, tt, 0.0), axis=0, keepdims=True)
    w2nd = jnp.exp(m2 - m1)
    out_ref[0, 0, :] = ((t1 + w2nd * t2) / z)[0, :]


def _tc_compute(ut_arr, it_arr, gt, ct, w1t, w2t, interpret=False):
    return pl.pallas_call(
        _tc_body,
        grid=(NB,),
        in_specs=[
            pl.BlockSpec((D, ROWS), lambda i: (0, i)),
            pl.BlockSpec((D, ROWS), lambda i: (0, i)),
            pl.BlockSpec((C_NUM, ROWS), lambda i: (0, i)),
            pl.BlockSpec((M, C_NUM), lambda i: (0, 0)),
            pl.BlockSpec((M, D), lambda i: (0, 0)),
            pl.BlockSpec((D, M), lambda i: (0, 0)),
        ],
        out_specs=pl.BlockSpec((1, 1, ROWS), lambda i: (i, 0, 0)),
        out_shape=jax.ShapeDtypeStruct((NB, 1, ROWS), jnp.float32),
        interpret=interpret,
    )(ut_arr, it_arr, gt, ct, w1t, w2t)


def kernel(x, uid_table, iid_table, C, w1, w2, gumbel):
    iu = x[:, 0].astype(jnp.int32)
    ii = x[:, 1].astype(jnp.int32)
    ut_arr, it_arr = _sc_gather(uid_table.T, iid_table.T, iu, ii)
    out = _tc_compute(ut_arr, it_arr, gumbel.T, C.T, w1.T, w2.T)
    return out.reshape(B)


# speedup vs baseline: 1.1152x; 1.1152x over previous
import functools

import jax
import jax.numpy as jnp
from jax import lax
from jax.experimental import pallas as pl
from jax.experimental.pallas import tpu as pltpu
from jax.experimental.pallas import tpu_sc as plsc

B = 16384
D = 32
M = 28
C_NUM = 64
TAU = 10.0

NUM_CORES = 2
NUM_SUBCORES = 16
NW = NUM_CORES * NUM_SUBCORES
BPW = B // NW

ROWS = 2048
NB = B // ROWS

PCOLS = 1000064
RB = 65536
RGRID = 16


def _repack_body(in_ref, out_ref):
    out_ref[...] = in_ref[...]


def _repack(tab_t):
    return pl.pallas_call(
        _repack_body,
        grid=(RGRID,),
        in_specs=[pl.BlockSpec((D, RB), lambda i: (0, i))],
        out_specs=pl.BlockSpec((D, RB), lambda i: (0, i)),
        out_shape=jax.ShapeDtypeStruct((D, PCOLS), jnp.float32),
    )(tab_t)


def _sc_gather_one(tab, idx):
    mesh = plsc.VectorSubcoreMesh(core_axis_name="c", subcore_axis_name="s")

    @functools.partial(
        pl.kernel,
        mesh=mesh,
        out_type=jax.ShapeDtypeStruct((D, B), jnp.float32),
        scratch_types=[
            pltpu.VMEM((BPW,), jnp.int32),
            pltpu.VMEM((D, BPW), jnp.float32),
            pltpu.SemaphoreType.DMA,
        ],
        compiler_params=pltpu.CompilerParams(use_tc_tiling_on_sc=False),
    )
    def k(tab_hbm, idx_hbm, out_hbm, idx_v, d_v, sem):
        wid = lax.axis_index("s") * NUM_CORES + lax.axis_index("c")
        base = wid * BPW
        pltpu.sync_copy(idx_hbm.at[pl.ds(base, BPW)], idx_v)
        copies = []
        for d in range(D):
            copies.append(pltpu.async_copy(tab_hbm.at[d].at[idx_v], d_v.at[d], sem))
        for c in copies:
            c.wait()
        pltpu.sync_copy(d_v, out_hbm.at[:, pl.ds(base, BPW)])

    return k(tab, idx)


def _tc_body(ut_ref, it_ref, gt_ref, ct_ref, w1t_ref, w2t_ref, out_ref):
    hi = lax.Precision.HIGHEST
    ut = ut_ref[...]
    it = it_ref[...]
    gt = gt_ref[...]
    ct = ct_ref[...]
    a = lax.dot_general(w1t_ref[...], ct, (((0,), (0,)), ((), ())),
                        precision=hi, preferred_element_type=jnp.float32)
    cw2t = lax.dot_general(w2t_ref[...], ct, (((1,), (0,)), ((), ())),
                           precision=hi, preferred_element_type=jnp.float32)
    st = lax.dot_general(a, ut, (((0,), (0,)), ((), ())),
                         precision=hi, preferred_element_type=jnp.float32)
    tt = lax.dot_general(cw2t, it, (((0,), (0,)), ((), ())),
                         precision=hi, preferred_element_type=jnp.float32)
    l = (st + gt) / TAU
    m1 = jnp.max(l, axis=0, keepdims=True)
    iota = lax.broadcasted_iota(jnp.int32, l.shape, 0)
    idx1 = jnp.min(jnp.where(l == m1, iota, C_NUM), axis=0, keepdims=True)
    oh1 = iota == idx1
    l2 = jnp.where(oh1, -jnp.inf, l)
    m2 = jnp.max(l2, axis=0, keepdims=True)
    idx2 = jnp.min(jnp.where(l2 == m2, iota, C_NUM), axis=0, keepdims=True)
    oh2 = iota == idx2
    z = jnp.sum(jnp.exp(l - m1), axis=0, keepdims=True)
    t1 = jnp.sum(jnp.where(oh1, tt, 0.0), axis=0, keepdims=True)
    t2 = jnp.sum(jnp.where(oh2, tt, 0.0), axis=0, keepdims=True)
    w2nd = jnp.exp(m2 - m1)
    out_ref[0, 0, :] = ((t1 + w2nd * t2) / z)[0, :]


def _tc_compute(ut_arr, it_arr, gt, ct, w1t, w2t, interpret=False):
    return pl.pallas_call(
        _tc_body,
        grid=(NB,),
        in_specs=[
            pl.BlockSpec((D, ROWS), lambda i: (0, i)),
            pl.BlockSpec((D, ROWS), lambda i: (0, i)),
            pl.BlockSpec((C_NUM, ROWS), lambda i: (0, i)),
            pl.BlockSpec((M, C_NUM), lambda i: (0, 0)),
            pl.BlockSpec((M, D), lambda i: (0, 0)),
            pl.BlockSpec((D, M), lambda i: (0, 0)),
        ],
        out_specs=pl.BlockSpec((1, 1, ROWS), lambda i: (i, 0, 0)),
        out_shape=jax.ShapeDtypeStruct((NB, 1, ROWS), jnp.float32),
        interpret=interpret,
    )(ut_arr, it_arr, gt, ct, w1t, w2t)


def kernel(x, uid_table, iid_table, C, w1, w2, gumbel):
    iu = x[:, 0].astype(jnp.int32)
    ii = x[:, 1].astype(jnp.int32)
    up = _repack(uid_table.T)
    ut_arr = _sc_gather_one(up, iu)
    ip = _repack(iid_table.T)
    it_arr = _sc_gather_one(ip, ii)
    out = _tc_compute(ut_arr, it_arr, gumbel.T, C.T, w1.T, w2.T)
    return out.reshape(B)


# speedup vs baseline: 22.7731x; 20.4208x over previous
import functools

import jax
import jax.numpy as jnp
from jax import lax
from jax.experimental import pallas as pl
from jax.experimental.pallas import tpu as pltpu
from jax.experimental.pallas import tpu_sc as plsc

B = 16384
D = 32
M = 28
C_NUM = 64
TAU = 10.0

NUM_CORES = 2
NUM_SUBCORES = 16
NW = NUM_CORES * NUM_SUBCORES
BPW = B // NW

ROWS = 2048
NB = B // ROWS

PCOLS = 1000448
RB = 65536
RGRID = 16


def _repack_body(in_ref, out_ref):
    out_ref[...] = in_ref[...].reshape(D, RB // 128, 128)


def _repack(tab_t):
    return pl.pallas_call(
        _repack_body,
        grid=(RGRID,),
        in_specs=[pl.BlockSpec((D, RB), lambda i: (0, i))],
        out_specs=pl.BlockSpec((D, RB // 128, 128), lambda i: (0, i, 0)),
        out_shape=jax.ShapeDtypeStruct((D, PCOLS // 128, 128), jnp.float32),
    )(tab_t)


def _sc_gather_one(tab, idx):
    mesh = plsc.VectorSubcoreMesh(core_axis_name="c", subcore_axis_name="s")

    @functools.partial(
        pl.kernel,
        mesh=mesh,
        out_type=jax.ShapeDtypeStruct((D, B), jnp.float32),
        scratch_types=[
            pltpu.VMEM((BPW,), jnp.int32),
            pltpu.VMEM((D, BPW), jnp.float32),
            pltpu.SemaphoreType.DMA,
        ],
        compiler_params=pltpu.CompilerParams(use_tc_tiling_on_sc=False),
    )
    def k(tab_hbm, idx_hbm, out_hbm, idx_v, d_v, sem):
        wid = lax.axis_index("s") * NUM_CORES + lax.axis_index("c")
        base = wid * BPW
        pltpu.sync_copy(idx_hbm.at[pl.ds(base, BPW)], idx_v)
        copies = []
        for d in range(D):
            copies.append(pltpu.async_copy(
                tab_hbm.at[pl.ds(d * PCOLS, PCOLS)].at[idx_v], d_v.at[d], sem))
        for c in copies:
            c.wait()
        pltpu.sync_copy(d_v, out_hbm.at[:, pl.ds(base, BPW)])

    return k(tab, idx)


def _tc_body(ut_ref, it_ref, gt_ref, ct_ref, w1t_ref, w2t_ref, out_ref):
    hi = lax.Precision.HIGHEST
    ut = ut_ref[...]
    it = it_ref[...]
    gt = gt_ref[...]
    ct = ct_ref[...]
    a = lax.dot_general(w1t_ref[...], ct, (((0,), (0,)), ((), ())),
                        precision=hi, preferred_element_type=jnp.float32)
    cw2t = lax.dot_general(w2t_ref[...], ct, (((1,), (0,)), ((), ())),
                           precision=hi, preferred_element_type=jnp.float32)
    st = lax.dot_general(a, ut, (((0,), (0,)), ((), ())),
                         precision=hi, preferred_element_type=jnp.float32)
    tt = lax.dot_general(cw2t, it, (((0,), (0,)), ((), ())),
                         precision=hi, preferred_element_type=jnp.float32)
    l = (st + gt) / TAU
    m1 = jnp.max(l, axis=0, keepdims=True)
    iota = lax.broadcasted_iota(jnp.int32, l.shape, 0)
    idx1 = jnp.min(jnp.where(l == m1, iota, C_NUM), axis=0, keepdims=True)
    oh1 = iota == idx1
    l2 = jnp.where(oh1, -jnp.inf, l)
    m2 = jnp.max(l2, axis=0, keepdims=True)
    idx2 = jnp.min(jnp.where(l2 == m2, iota, C_NUM), axis=0, keepdims=True)
    oh2 = iota == idx2
    z = jnp.sum(jnp.exp(l - m1), axis=0, keepdims=True)
    t1 = jnp.sum(jnp.where(oh1, tt, 0.0), axis=0, keepdims=True)
    t2 = jnp.sum(jnp.where(oh2, tt, 0.0), axis=0, keepdims=True)
    w2nd = jnp.exp(m2 - m1)
    out_ref[0, 0, :] = ((t1 + w2nd * t2) / z)[0, :]


def _tc_compute(ut_arr, it_arr, gt, ct, w1t, w2t, interpret=False):
    return pl.pallas_call(
        _tc_body,
        grid=(NB,),
        in_specs=[
            pl.BlockSpec((D, ROWS), lambda i: (0, i)),
            pl.BlockSpec((D, ROWS), lambda i: (0, i)),
            pl.BlockSpec((C_NUM, ROWS), lambda i: (0, i)),
            pl.BlockSpec((M, C_NUM), lambda i: (0, 0)),
            pl.BlockSpec((M, D), lambda i: (0, 0)),
            pl.BlockSpec((D, M), lambda i: (0, 0)),
        ],
        out_specs=pl.BlockSpec((1, 1, ROWS), lambda i: (i, 0, 0)),
        out_shape=jax.ShapeDtypeStruct((NB, 1, ROWS), jnp.float32),
        interpret=interpret,
    )(ut_arr, it_arr, gt, ct, w1t, w2t)


def kernel(x, uid_table, iid_table, C, w1, w2, gumbel):
    iu = x[:, 0].astype(jnp.int32)
    ii = x[:, 1].astype(jnp.int32)
    up = _repack(uid_table.T).reshape(D * PCOLS)
    ut_arr = _sc_gather_one(up, iu)
    ip = _repack(iid_table.T).reshape(D * PCOLS)
    it_arr = _sc_gather_one(ip, ii)
    out = _tc_compute(ut_arr, it_arr, gumbel.T, C.T, w1.T, w2.T)
    return out.reshape(B)


# speedup vs baseline: 22.8602x; 1.0038x over previous
import functools

import jax
import jax.numpy as jnp
from jax import lax
from jax.experimental import pallas as pl
from jax.experimental.pallas import tpu as pltpu
from jax.experimental.pallas import tpu_sc as plsc

B = 16384
D = 32
M = 28
C_NUM = 64
TAU = 10.0

NUM_CORES = 2
NUM_SUBCORES = 16
NW = NUM_CORES * NUM_SUBCORES
BPW = B // NW

ROWS = 4096
NB = B // ROWS

PCOLS = 1000448
RB = 65536
RGRID = 16


def _repack_body(in_ref, out_ref):
    out_ref[...] = in_ref[...].reshape(D, RB // 128, 128)


def _repack(tab_t):
    return pl.pallas_call(
        _repack_body,
        grid=(RGRID,),
        in_specs=[pl.BlockSpec((D, RB), lambda i: (0, i))],
        out_specs=pl.BlockSpec((D, RB // 128, 128), lambda i: (0, i, 0)),
        out_shape=jax.ShapeDtypeStruct((D, PCOLS // 128, 128), jnp.float32),
    )(tab_t)


def _sc_gather_one(tab, idx):
    mesh = plsc.VectorSubcoreMesh(core_axis_name="c", subcore_axis_name="s")

    @functools.partial(
        pl.kernel,
        mesh=mesh,
        out_type=jax.ShapeDtypeStruct((D, B), jnp.float32),
        scratch_types=[
            pltpu.VMEM((BPW,), jnp.int32),
            pltpu.VMEM((D, BPW), jnp.float32),
            pltpu.SemaphoreType.DMA,
        ],
        compiler_params=pltpu.CompilerParams(use_tc_tiling_on_sc=False),
    )
    def k(tab_hbm, idx_hbm, out_hbm, idx_v, d_v, sem):
        wid = lax.axis_index("s") * NUM_CORES + lax.axis_index("c")
        base = wid * BPW
        pltpu.sync_copy(idx_hbm.at[pl.ds(base, BPW)], idx_v)
        copies = []
        for d in range(D):
            copies.append(pltpu.async_copy(
                tab_hbm.at[pl.ds(d * PCOLS, PCOLS)].at[idx_v], d_v.at[d], sem))
        for c in copies:
            c.wait()
        pltpu.sync_copy(d_v, out_hbm.at[:, pl.ds(base, BPW)])

    return k(tab, idx)


def _tc_body(ut_ref, it_ref, gt_ref, ct_ref, w1t_ref, w2t_ref, out_ref):
    hi = lax.Precision.HIGHEST
    ut = ut_ref[...]
    it = it_ref[...]
    gt = gt_ref[...]
    ct = ct_ref[...]
    a = lax.dot_general(w1t_ref[...], ct, (((0,), (0,)), ((), ())),
                        precision=hi, preferred_element_type=jnp.float32)
    cw2t = lax.dot_general(w2t_ref[...], ct, (((1,), (0,)), ((), ())),
                           precision=hi, preferred_element_type=jnp.float32)
    st = lax.dot_general(a, ut, (((0,), (0,)), ((), ())),
                         precision=hi, preferred_element_type=jnp.float32)
    tt = lax.dot_general(cw2t, it, (((0,), (0,)), ((), ())),
                         precision=hi, preferred_element_type=jnp.float32)
    l = (st + gt) / TAU
    m1 = jnp.max(l, axis=0, keepdims=True)
    iota = lax.broadcasted_iota(jnp.int32, l.shape, 0)
    idx1 = jnp.min(jnp.where(l == m1, iota, C_NUM), axis=0, keepdims=True)
    oh1 = iota == idx1
    l2 = jnp.where(oh1, -jnp.inf, l)
    m2 = jnp.max(l2, axis=0, keepdims=True)
    idx2 = jnp.min(jnp.where(l2 == m2, iota, C_NUM), axis=0, keepdims=True)
    oh2 = iota == idx2
    z = jnp.sum(jnp.exp(l - m1), axis=0, keepdims=True)
    t1 = jnp.sum(jnp.where(oh1, tt, 0.0), axis=0, keepdims=True)
    t2 = jnp.sum(jnp.where(oh2, tt, 0.0), axis=0, keepdims=True)
    w2nd = jnp.exp(m2 - m1)
    out_ref[0, 0, :] = ((t1 + w2nd * t2) / z)[0, :]


def _tc_compute(ut_arr, it_arr, gt, ct, w1t, w2t, interpret=False):
    return pl.pallas_call(
        _tc_body,
        grid=(NB,),
        in_specs=[
            pl.BlockSpec((D, ROWS), lambda i: (0, i)),
            pl.BlockSpec((D, ROWS), lambda i: (0, i)),
            pl.BlockSpec((C_NUM, ROWS), lambda i: (0, i)),
            pl.BlockSpec((M, C_NUM), lambda i: (0, 0)),
            pl.BlockSpec((M, D), lambda i: (0, 0)),
            pl.BlockSpec((D, M), lambda i: (0, 0)),
        ],
        out_specs=pl.BlockSpec((1, 1, ROWS), lambda i: (i, 0, 0)),
        out_shape=jax.ShapeDtypeStruct((NB, 1, ROWS), jnp.float32),
        interpret=interpret,
    )(ut_arr, it_arr, gt, ct, w1t, w2t)


def kernel(x, uid_table, iid_table, C, w1, w2, gumbel):
    iu = x[:, 0].astype(jnp.int32)
    ii = x[:, 1].astype(jnp.int32)
    up = _repack(uid_table.T).reshape(D * PCOLS)
    ut_arr = _sc_gather_one(up, iu)
    ip = _repack(iid_table.T).reshape(D * PCOLS)
    it_arr = _sc_gather_one(ip, ii)
    out = _tc_compute(ut_arr, it_arr, gumbel.T, C.T, w1.T, w2.T)
    return out.reshape(B)


# speedup vs baseline: 22.8727x; 1.0005x over previous
import functools

import jax
import jax.numpy as jnp
from jax import lax
from jax.experimental import pallas as pl
from jax.experimental.pallas import tpu as pltpu
from jax.experimental.pallas import tpu_sc as plsc

B = 16384
D = 32
M = 28
C_NUM = 64
TAU = 10.0

NUM_CORES = 2
NUM_SUBCORES = 16
NW = NUM_CORES * NUM_SUBCORES
BPW = B // NW

ROWS = 4096
NB = B // ROWS

PCOLS = 1000448
RB = 65536
RGRID = 16


def _repack(tab_t, nd=D, dblk=0):
    def body(in_ref, out_ref):
        out_ref[...] = in_ref[...].reshape(nd, RB // 128, 128)

    return pl.pallas_call(
        body,
        grid=(RGRID,),
        in_specs=[pl.BlockSpec((nd, RB), lambda i: (dblk, i))],
        out_specs=pl.BlockSpec((nd, RB // 128, 128), lambda i: (0, i, 0)),
        out_shape=jax.ShapeDtypeStruct((nd, PCOLS // 128, 128), jnp.float32),
    )(tab_t)


def _sc_gather_one(tab, idx, nd=D):
    mesh = plsc.VectorSubcoreMesh(core_axis_name="c", subcore_axis_name="s")

    @functools.partial(
        pl.kernel,
        mesh=mesh,
        out_type=jax.ShapeDtypeStruct((nd, B), jnp.float32),
        scratch_types=[
            pltpu.VMEM((BPW,), jnp.int32),
            pltpu.VMEM((nd, BPW), jnp.float32),
            pltpu.SemaphoreType.DMA,
        ],
        compiler_params=pltpu.CompilerParams(use_tc_tiling_on_sc=False),
    )
    def k(tab_hbm, idx_hbm, out_hbm, idx_v, d_v, sem):
        wid = lax.axis_index("s") * NUM_CORES + lax.axis_index("c")
        base = wid * BPW
        pltpu.sync_copy(idx_hbm.at[pl.ds(base, BPW)], idx_v)
        copies = []
        for d in range(nd):
            copies.append(pltpu.async_copy(
                tab_hbm.at[pl.ds(d * PCOLS, PCOLS)].at[idx_v], d_v.at[d], sem))
        for c in copies:
            c.wait()
        pltpu.sync_copy(d_v, out_hbm.at[:, pl.ds(base, BPW)])

    return k(tab, idx)


def _tc_body(ut_ref, ita_ref, itb_ref, gt_ref, ct_ref, w1t_ref, w2t_ref,
             out_ref):
    hi = lax.Precision.HIGHEST
    ut = ut_ref[...]
    gt = gt_ref[...]
    ct = ct_ref[...]
    a = lax.dot_general(w1t_ref[...], ct, (((0,), (0,)), ((), ())),
                        precision=hi, preferred_element_type=jnp.float32)
    cw2t = lax.dot_general(w2t_ref[...], ct, (((1,), (0,)), ((), ())),
                           precision=hi, preferred_element_type=jnp.float32)
    st = lax.dot_general(a, ut, (((0,), (0,)), ((), ())),
                         precision=hi, preferred_element_type=jnp.float32)
    tt = (lax.dot_general(cw2t[:D // 2], ita_ref[...], (((0,), (0,)), ((), ())),
                          precision=hi, preferred_element_type=jnp.float32)
          + lax.dot_general(cw2t[D // 2:], itb_ref[...], (((0,), (0,)), ((), ())),
                            precision=hi, preferred_element_type=jnp.float32))
    l = (st + gt) / TAU
    m1 = jnp.max(l, axis=0, keepdims=True)
    iota = lax.broadcasted_iota(jnp.int32, l.shape, 0)
    idx1 = jnp.min(jnp.where(l == m1, iota, C_NUM), axis=0, keepdims=True)
    oh1 = iota == idx1
    l2 = jnp.where(oh1, -jnp.inf, l)
    m2 = jnp.max(l2, axis=0, keepdims=True)
    idx2 = jnp.min(jnp.where(l2 == m2, iota, C_NUM), axis=0, keepdims=True)
    oh2 = iota == idx2
    z = jnp.sum(jnp.exp(l - m1), axis=0, keepdims=True)
    t1 = jnp.sum(jnp.where(oh1, tt, 0.0), axis=0, keepdims=True)
    t2 = jnp.sum(jnp.where(oh2, tt, 0.0), axis=0, keepdims=True)
    w2nd = jnp.exp(m2 - m1)
    out_ref[0, 0, :] = ((t1 + w2nd * t2) / z)[0, :]


def _tc_compute(ut_arr, ita_arr, itb_arr, gt, ct, w1t, w2t, interpret=False):
    return pl.pallas_call(
        _tc_body,
        grid=(NB,),
        in_specs=[
            pl.BlockSpec((D, ROWS), lambda i: (0, i)),
            pl.BlockSpec((D // 2, ROWS), lambda i: (0, i)),
            pl.BlockSpec((D // 2, ROWS), lambda i: (0, i)),
            pl.BlockSpec((C_NUM, ROWS), lambda i: (0, i)),
            pl.BlockSpec((M, C_NUM), lambda i: (0, 0)),
            pl.BlockSpec((M, D), lambda i: (0, 0)),
            pl.BlockSpec((D, M), lambda i: (0, 0)),
        ],
        out_specs=pl.BlockSpec((1, 1, ROWS), lambda i: (i, 0, 0)),
        out_shape=jax.ShapeDtypeStruct((NB, 1, ROWS), jnp.float32),
        interpret=interpret,
    )(ut_arr, ita_arr, itb_arr, gt, ct, w1t, w2t)


def kernel(x, uid_table, iid_table, C, w1, w2, gumbel):
    iu = x[:, 0].astype(jnp.int32)
    ii = x[:, 1].astype(jnp.int32)
    up = _repack(uid_table.T).reshape(D * PCOLS)
    ut_arr = _sc_gather_one(up, iu)
    ipa = _repack(iid_table.T, nd=D // 2, dblk=0).reshape(D // 2 * PCOLS)
    ita_arr = _sc_gather_one(ipa, ii, nd=D // 2)
    ipb = _repack(iid_table.T, nd=D // 2, dblk=1).reshape(D // 2 * PCOLS)
    itb_arr = _sc_gather_one(ipb, ii, nd=D // 2)
    out = _tc_compute(ut_arr, ita_arr, itb_arr, gumbel.T, C.T, w1.T, w2.T)
    return out.reshape(B)
